# Initial kernel scaffold; baseline (speedup 1.0000x reference)
import jax
import jax.numpy as jnp
from jax.experimental import pallas as pl


def kernel(x, gate_W, gate_b, W1, b1, W2, b2):
    raise NotImplementedError("write your pallas kernel here")


# speedup vs baseline: 2.2714x; 2.2714x over previous
import functools
import math

import jax
import jax.numpy as jnp
from jax.experimental import pallas as pl
from jax.experimental.pallas import tpu as pltpu

D_MODEL = 1024
NUM_EXPERTS = 8
HIDDEN = 4096
TOKENS = 2048
ASSIGN = 2 * TOKENS
EPAD = 128
BM = 128
NB = ASSIGN // BM + NUM_EXPERTS
NPAD = NB * BM

_INTERPRET = False


def _gate_kernel(x_ref, gw_ref, gb_ref, i0_ref, i1_ref, w0_ref, w1_ref,
                 ll_ref):
    x = x_ref[...]
    gw = gw_ref[...]
    logits = jnp.dot(x, gw, preferred_element_type=jnp.float32)
    logits = logits + gb_ref[...]
    lane = jax.lax.broadcasted_iota(jnp.int32, (TOKENS, EPAD), 1)
    valid = lane < NUM_EXPERTS
    logits = jnp.where(valid, logits, jnp.float32(-1e30))
    m = jnp.max(logits, axis=1, keepdims=True)
    p = jnp.where(valid, jnp.exp(logits - m), 0.0)
    s = jnp.sum(p, axis=1, keepdims=True)
    probs = p / s
    big = jnp.int32(EPAD)
    v0 = jnp.max(probs, axis=1, keepdims=True)
    i0 = jnp.min(jnp.where(probs == v0, lane, big), axis=1, keepdims=True)
    probs1 = jnp.where(lane == i0, jnp.float32(-1.0), probs)
    v1 = jnp.max(probs1, axis=1, keepdims=True)
    i1 = jnp.min(jnp.where(probs1 == v1, lane, big), axis=1, keepdims=True)
    denom = v0 + v1 + jnp.float32(1e-9)
    w0 = v0 / denom
    w1 = v1 / denom
    i0_ref[...] = jnp.broadcast_to(i0, (TOKENS, EPAD))
    i1_ref[...] = jnp.broadcast_to(i1, (TOKENS, EPAD))
    w0_ref[...] = jnp.broadcast_to(w0, (TOKENS, EPAD))
    w1_ref[...] = jnp.broadcast_to(w1, (TOKENS, EPAD))
    pm = jnp.sum(probs, axis=0, keepdims=True) / jnp.float32(TOKENS)
    diff = jnp.where(valid[0:1, :], pm - jnp.float32(1.0 / NUM_EXPERTS), 0.0)
    ll = jnp.sum(diff * diff) / jnp.float32(NUM_EXPERTS)
    ll_ref[...] = jnp.full((1, EPAD), ll, dtype=jnp.float32)


def _gate(x2d, gate_W, gate_b):
    gwp = jnp.zeros((D_MODEL, EPAD), jnp.float32).at[:, :NUM_EXPERTS].set(
        gate_W)
    gbp = jnp.zeros((1, EPAD), jnp.float32).at[0, :NUM_EXPERTS].set(gate_b)
    outs = pl.pallas_call(
        _gate_kernel,
        out_shape=(
            jax.ShapeDtypeStruct((TOKENS, EPAD), jnp.int32),
            jax.ShapeDtypeStruct((TOKENS, EPAD), jnp.int32),
            jax.ShapeDtypeStruct((TOKENS, EPAD), jnp.float32),
            jax.ShapeDtypeStruct((TOKENS, EPAD), jnp.float32),
            jax.ShapeDtypeStruct((1, EPAD), jnp.float32),
        ),
        interpret=_INTERPRET,
    )(x2d, gwp, gbp)
    return outs


def _ffn_kernel(be_ref, xs_ref, w1_ref, b1_ref, w2_ref, b2_ref, out_ref):
    xb = xs_ref[...]
    h = jnp.dot(xb, w1_ref[0], preferred_element_type=jnp.float32)
    h = h + b1_ref[0]
    h = 0.5 * h * (1.0 + jax.lax.erf(h * jnp.float32(1.0 / math.sqrt(2.0))))
    out = jnp.dot(h.astype(jnp.bfloat16), w2_ref[0],
                  preferred_element_type=jnp.float32)
    out_ref[...] = out + b2_ref[0]


def _grouped_ffn(block_expert, xs, W1, b1, W2, b2):
    grid_spec = pltpu.PrefetchScalarGridSpec(
        num_scalar_prefetch=1,
        grid=(NB,),
        in_specs=[
            pl.BlockSpec((BM, D_MODEL), lambda b, be: (b, 0)),
            pl.BlockSpec((1, D_MODEL, HIDDEN), lambda b, be: (be[b], 0, 0)),
            pl.BlockSpec((1, 1, HIDDEN), lambda b, be: (be[b], 0, 0)),
            pl.BlockSpec((1, HIDDEN, D_MODEL), lambda b, be: (be[b], 0, 0)),
            pl.BlockSpec((1, 1, D_MODEL), lambda b, be: (be[b], 0, 0)),
        ],
        out_specs=pl.BlockSpec((BM, D_MODEL), lambda b, be: (b, 0)),
    )
    return pl.pallas_call(
        _ffn_kernel,
        grid_spec=grid_spec,
        out_shape=jax.ShapeDtypeStruct((NPAD, D_MODEL), jnp.float32),
        compiler_params=pltpu.CompilerParams(
            dimension_semantics=("arbitrary",)),
        interpret=_INTERPRET,
    )(block_expert, xs.astype(jnp.bfloat16), W1.astype(jnp.bfloat16),
      b1.reshape(NUM_EXPERTS, 1, HIDDEN), W2.astype(jnp.bfloat16),
      b2.reshape(NUM_EXPERTS, 1, D_MODEL))


def _route(e_flat):
    oh = (e_flat[:, None] == jnp.arange(NUM_EXPERTS)[None, :]).astype(
        jnp.int32)
    counts = oh.sum(axis=0)
    padded = ((counts + BM - 1) // BM) * BM
    offsets = jnp.concatenate(
        [jnp.zeros((1,), jnp.int32), jnp.cumsum(padded)[:-1]]).astype(
            jnp.int32)
    rank = jnp.cumsum(oh, axis=0) - oh
    rank_a = jnp.take_along_axis(rank, e_flat[:, None], axis=1)[:, 0]
    pos = (offsets[e_flat] + rank_a).astype(jnp.int32)
    tok = jnp.arange(ASSIGN, dtype=jnp.int32) // 2
    row_token = jnp.zeros((NPAD,), jnp.int32).at[pos].set(tok)
    off_blk = offsets // BM
    block_expert = (jnp.arange(NB, dtype=jnp.int32)[:, None]
                    >= off_blk[None, :]).astype(jnp.int32).sum(axis=1) - 1
    return pos, row_token, block_expert.astype(jnp.int32)


def kernel(x, gate_W, gate_b, W1, b1, W2, b2):
    bsz, seq_len, d_model = x.shape
    x2d = x.reshape(TOKENS, D_MODEL)
    i0b, i1b, w0b, w1b, llb = _gate(x2d, gate_W, gate_b)
    i0 = i0b[:, 0]
    i1 = i1b[:, 0]
    e_flat = jnp.stack([i0, i1], axis=1).reshape(ASSIGN)
    w_flat = jnp.stack([w0b[:, 0], w1b[:, 0]], axis=1).reshape(ASSIGN)

    pos, row_token, block_expert = _route(e_flat)
    xs = x2d[row_token]
    rows = _grouped_ffn(block_expert, xs, W1, b1, W2, b2)
    gathered = rows[pos].reshape(TOKENS, 2, D_MODEL)
    moe = (gathered * w_flat.reshape(TOKENS, 2, 1)).sum(axis=1)
    moe_out = moe.reshape(bsz, seq_len, d_model)
    load_loss = llb[0, 0].reshape(())
    return moe_out, load_loss


# speedup vs baseline: 2.5213x; 1.1101x over previous
import functools
import math

import jax
import jax.numpy as jnp
from jax.experimental import pallas as pl
from jax.experimental.pallas import tpu as pltpu

D_MODEL = 1024
NUM_EXPERTS = 8
HIDDEN = 4096
TOKENS = 2048
ASSIGN = 2 * TOKENS
EPAD = 128
BM = 128
NB = ASSIGN // BM + NUM_EXPERTS
NPAD = NB * BM

_INTERPRET = False


def _gate_kernel(x_ref, gw_ref, gb_ref, i0_ref, i1_ref, w0_ref, w1_ref,
                 ll_ref):
    x = x_ref[...]
    gw = gw_ref[...]
    logits = jnp.dot(x, gw, preferred_element_type=jnp.float32)
    logits = logits + gb_ref[...]
    lane = jax.lax.broadcasted_iota(jnp.int32, (TOKENS, EPAD), 1)
    valid = lane < NUM_EXPERTS
    logits = jnp.where(valid, logits, jnp.float32(-1e30))
    m = jnp.max(logits, axis=1, keepdims=True)
    p = jnp.where(valid, jnp.exp(logits - m), 0.0)
    s = jnp.sum(p, axis=1, keepdims=True)
    probs = p / s
    big = jnp.int32(EPAD)
    v0 = jnp.max(probs, axis=1, keepdims=True)
    i0 = jnp.min(jnp.where(probs == v0, lane, big), axis=1, keepdims=True)
    probs1 = jnp.where(lane == i0, jnp.float32(-1.0), probs)
    v1 = jnp.max(probs1, axis=1, keepdims=True)
    i1 = jnp.min(jnp.where(probs1 == v1, lane, big), axis=1, keepdims=True)
    denom = v0 + v1 + jnp.float32(1e-9)
    w0 = v0 / denom
    w1 = v1 / denom
    i0_ref[...] = jnp.broadcast_to(i0, (TOKENS, EPAD))
    i1_ref[...] = jnp.broadcast_to(i1, (TOKENS, EPAD))
    w0_ref[...] = jnp.broadcast_to(w0, (TOKENS, EPAD))
    w1_ref[...] = jnp.broadcast_to(w1, (TOKENS, EPAD))
    pm = jnp.sum(probs, axis=0, keepdims=True) / jnp.float32(TOKENS)
    diff = jnp.where(valid[0:1, :], pm - jnp.float32(1.0 / NUM_EXPERTS), 0.0)
    ll = jnp.sum(diff * diff) / jnp.float32(NUM_EXPERTS)
    ll_ref[...] = jnp.full((1, EPAD), ll, dtype=jnp.float32)


def _gate(x2d, gate_W, gate_b):
    gwp = jnp.zeros((D_MODEL, EPAD), jnp.float32).at[:, :NUM_EXPERTS].set(
        gate_W)
    gbp = jnp.zeros((1, EPAD), jnp.float32).at[0, :NUM_EXPERTS].set(gate_b)
    outs = pl.pallas_call(
        _gate_kernel,
        out_shape=(
            jax.ShapeDtypeStruct((TOKENS, EPAD), jnp.int32),
            jax.ShapeDtypeStruct((TOKENS, EPAD), jnp.int32),
            jax.ShapeDtypeStruct((TOKENS, EPAD), jnp.float32),
            jax.ShapeDtypeStruct((TOKENS, EPAD), jnp.float32),
            jax.ShapeDtypeStruct((1, EPAD), jnp.float32),
        ),
        interpret=_INTERPRET,
    )(x2d, gwp, gbp)
    return outs


NH = 2
BH = HIDDEN // NH


def _ffn_kernel(be_ref, xs_ref, w1_ref, b1_ref, w2_ref, b2_ref, out_ref,
                w1bf_ref, w2bf_ref, acc_ref):
    h = pl.program_id(0)
    b = pl.program_id(1)
    prev = be_ref[jnp.maximum(b - 1, 0)]
    changed = jnp.logical_or(b == 0, be_ref[b] != prev)

    @pl.when(changed)
    def _cast():
        w1bf_ref[...] = w1_ref[0].astype(jnp.bfloat16)
        w2bf_ref[...] = w2_ref[0].astype(jnp.bfloat16)

    xb = xs_ref[...]
    hm = jnp.dot(xb, w1bf_ref[...], preferred_element_type=jnp.float32)
    hm = hm + b1_ref[0]
    hm = 0.5 * hm * (1.0 + jax.lax.erf(hm * jnp.float32(1.0 / math.sqrt(2.0))))
    part = jnp.dot(hm.astype(jnp.bfloat16), w2bf_ref[...],
                   preferred_element_type=jnp.float32)

    @pl.when(h == 0)
    def _store():
        acc_ref[pl.ds(b * BM, BM), :] = (part + b2_ref[0]).astype(
            jnp.bfloat16)

    @pl.when(h == NH - 1)
    def _final():
        out_ref[...] = part + acc_ref[pl.ds(b * BM, BM), :].astype(
            jnp.float32)


def _grouped_ffn(block_expert, xs, W1, b1, W2, b2):
    grid_spec = pltpu.PrefetchScalarGridSpec(
        num_scalar_prefetch=1,
        grid=(NH, NB),
        in_specs=[
            pl.BlockSpec((BM, D_MODEL), lambda h, b, be: (b, 0)),
            pl.BlockSpec((1, D_MODEL, BH), lambda h, b, be: (be[b], 0, h)),
            pl.BlockSpec((1, 1, BH), lambda h, b, be: (be[b], 0, h)),
            pl.BlockSpec((1, BH, D_MODEL), lambda h, b, be: (be[b], h, 0)),
            pl.BlockSpec((1, 1, D_MODEL), lambda h, b, be: (be[b], 0, 0)),
        ],
        out_specs=pl.BlockSpec((BM, D_MODEL), lambda h, b, be: (b, 0)),
        scratch_shapes=[
            pltpu.VMEM((D_MODEL, BH), jnp.bfloat16),
            pltpu.VMEM((BH, D_MODEL), jnp.bfloat16),
            pltpu.VMEM((NPAD, D_MODEL), jnp.bfloat16),
        ],
    )
    return pl.pallas_call(
        _ffn_kernel,
        grid_spec=grid_spec,
        out_shape=jax.ShapeDtypeStruct((NPAD, D_MODEL), jnp.float32),
        compiler_params=pltpu.CompilerParams(
            dimension_semantics=("arbitrary", "arbitrary")),
        interpret=_INTERPRET,
    )(block_expert, xs, W1,
      b1.reshape(NUM_EXPERTS, 1, HIDDEN), W2,
      b2.reshape(NUM_EXPERTS, 1, D_MODEL))


def _route(e_flat):
    oh = (e_flat[:, None] == jnp.arange(NUM_EXPERTS)[None, :]).astype(
        jnp.int32)
    counts = oh.sum(axis=0)
    padded = ((counts + BM - 1) // BM) * BM
    offsets = jnp.concatenate(
        [jnp.zeros((1,), jnp.int32), jnp.cumsum(padded)[:-1]]).astype(
            jnp.int32)
    rank = jnp.cumsum(oh, axis=0) - oh
    rank_a = jnp.take_along_axis(rank, e_flat[:, None], axis=1)[:, 0]
    pos = (offsets[e_flat] + rank_a).astype(jnp.int32)
    tok = jnp.arange(ASSIGN, dtype=jnp.int32) // 2
    row_token = jnp.zeros((NPAD,), jnp.int32).at[pos].set(tok)
    off_blk = offsets // BM
    block_expert = (jnp.arange(NB, dtype=jnp.int32)[:, None]
                    >= off_blk[None, :]).astype(jnp.int32).sum(axis=1) - 1
    return pos, row_token, block_expert.astype(jnp.int32)


def kernel(x, gate_W, gate_b, W1, b1, W2, b2):
    bsz, seq_len, d_model = x.shape
    x2d = x.reshape(TOKENS, D_MODEL)
    i0b, i1b, w0b, w1b, llb = _gate(x2d, gate_W, gate_b)
    i0 = i0b[:, 0]
    i1 = i1b[:, 0]
    e_flat = jnp.stack([i0, i1], axis=1).reshape(ASSIGN)
    w_flat = jnp.stack([w0b[:, 0], w1b[:, 0]], axis=1).reshape(ASSIGN)

    pos, row_token, block_expert = _route(e_flat)
    xs = x2d.astype(jnp.bfloat16)[row_token]
    rows = _grouped_ffn(block_expert, xs, W1, b1, W2, b2)
    gathered = rows[pos].reshape(TOKENS, 2, D_MODEL)
    moe = (gathered * w_flat.reshape(TOKENS, 2, 1)).sum(axis=1)
    moe_out = moe.reshape(bsz, seq_len, d_model)
    load_loss = llb[0, 0].reshape(())
    return moe_out, load_loss
